# Initial kernel scaffold; baseline (speedup 1.0000x reference)
import functools

import jax
import jax.numpy as jnp
from jax import lax
from jax.experimental import pallas as pl
from jax.experimental.pallas import tpu as pltpu
from jax.experimental.pallas import tpu_sc as plsc

_NS = 16
_H = 256
_BLK = 1000
_G = 128


def _sc_segment_add(hs_flat, src, dst, n, w, e):
    tpe = e // _NS
    b = 80
    nb = tpe // b
    rpt = n // _NS
    zr = 16
    nz_full = rpt // zr
    nz_rem = rpt - nz_full * zr

    mesh = plsc.VectorSubcoreMesh(core_axis_name="c", subcore_axis_name="s")

    @functools.partial(
        pl.kernel,
        out_type=jax.ShapeDtypeStruct((2 * n, w), jnp.float32),
        mesh=mesh,
        scratch_types=[
            pltpu.VMEM((b,), jnp.int32),
            pltpu.VMEM((b,), jnp.int32),
            pltpu.VMEM((b, w), jnp.float32),
            pltpu.VMEM((zr, w), jnp.float32),
            pltpu.VMEM_SHARED((n, w), jnp.float32),
            pltpu.SemaphoreType.DMA,
        ],
    )
    def k(hs_hbm, src_hbm, dst_hbm, out_hbm, gidx_v, dst_v, rows_v, zbuf, acc,
          sem):
        c = lax.axis_index("c")
        s = lax.axis_index("s")
        zero16 = jnp.zeros((16,), jnp.float32)
        for r in range(zr):
            for q in range(w // 16):
                zbuf[r, pl.ds(q * 16, 16)] = zero16
        rbase = s * rpt

        def zloop(j, carry):
            pltpu.sync_copy(zbuf, acc.at[pl.ds(rbase + j * zr, zr)])
            return carry

        lax.fori_loop(0, nz_full, zloop, 0)
        if nz_rem:
            pltpu.sync_copy(zbuf.at[pl.ds(0, nz_rem)],
                            acc.at[pl.ds(rbase + nz_full * zr, nz_rem)])
        plsc.subcore_barrier()

        ebase = s * tpe
        coff = c * n

        def eloop(i, carry):
            off = ebase + i * b
            pltpu.sync_copy(src_hbm.at[pl.ds(off, b)], gidx_v)
            pltpu.sync_copy(dst_hbm.at[pl.ds(off, b)], dst_v)
            for q in range(b // 16):
                sl = pl.ds(q * 16, 16)
                gidx_v[sl] = gidx_v[sl] + coff
            pltpu.async_copy(hs_hbm.at[gidx_v], rows_v, sem).wait()
            pltpu.sync_copy(rows_v, acc.at[dst_v], add=True)
            return carry

        lax.fori_loop(0, nb, eloop, 0)
        plsc.subcore_barrier()
        pltpu.sync_copy(acc.at[pl.ds(rbase, rpt)],
                        out_hbm.at[pl.ds(coff + rbase, rpt)])

    return k(hs_flat, src, dst)


def _tc_mlp_stats(hs, aggs, e_row, w1a, w1b, b1, w2, b2, n, w):
    nblk = n // _BLK

    def body(hs_ref, agg_ref, e_ref, w1a_ref, w1b_ref, b1_ref, w2_ref, b2_ref,
             t_ref, stats_ref):
        z0 = hs_ref[0] * e_ref[...] + agg_ref[0]
        z1 = hs_ref[1] * e_ref[...] + agg_ref[1]
        u = jnp.dot(z0, w1a_ref[...], preferred_element_type=jnp.float32)
        u = u + jnp.dot(z1, w1b_ref[...], preferred_element_type=jnp.float32)
        a = jnp.maximum(u + b1_ref[...], 0.0)
        t = jnp.dot(a, w2_ref[...], preferred_element_type=jnp.float32)
        t = t + b2_ref[...]
        t_ref[...] = t

        @pl.when(pl.program_id(0) == 0)
        def _():
            stats_ref[...] = jnp.zeros_like(stats_ref)

        stats_ref[0:1, :] += jnp.sum(t, axis=0, keepdims=True)
        stats_ref[1:2, :] += jnp.sum(t * t, axis=0, keepdims=True)

    return pl.pallas_call(
        body,
        grid=(nblk,),
        in_specs=[
            pl.BlockSpec((2, _BLK, w), lambda i: (0, i, 0)),
            pl.BlockSpec((2, _BLK, w), lambda i: (0, i, 0)),
            pl.BlockSpec((1, w), lambda i: (0, 0)),
            pl.BlockSpec((w, _H), lambda i: (0, 0)),
            pl.BlockSpec((w, _H), lambda i: (0, 0)),
            pl.BlockSpec((1, _H), lambda i: (0, 0)),
            pl.BlockSpec((_H, _H), lambda i: (0, 0)),
            pl.BlockSpec((1, _H), lambda i: (0, 0)),
        ],
        out_specs=[
            pl.BlockSpec((_BLK, _H), lambda i: (i, 0)),
            pl.BlockSpec((2, _H), lambda i: (0, 0)),
        ],
        out_shape=[
            jax.ShapeDtypeStruct((n, _H), jnp.float32),
            jax.ShapeDtypeStruct((2, _H), jnp.float32),
        ],
    )(hs, aggs, e_row, w1a, w1b, b1, w2, b2)


def _tc_bn_relu(t, stats, gamma, beta, n):
    nblk = n // _BLK
    inv_n = 1.0 / n

    def body(t_ref, stats_ref, gamma_ref, beta_ref, out_ref):
        mu = stats_ref[0:1, :] * inv_n
        var = stats_ref[1:2, :] * inv_n - mu * mu
        inv = lax.rsqrt(var + 1e-5)
        scale = gamma_ref[...] * inv
        shift = beta_ref[...] - mu * scale
        hnew = jnp.maximum(t_ref[...] * scale + shift, 0.0)
        out_ref[0] = hnew[:, :128]
        out_ref[1] = hnew[:, 128:]

    return pl.pallas_call(
        body,
        grid=(nblk,),
        in_specs=[
            pl.BlockSpec((_BLK, _H), lambda i: (i, 0)),
            pl.BlockSpec((2, _H), lambda i: (0, 0)),
            pl.BlockSpec((1, _H), lambda i: (0, 0)),
            pl.BlockSpec((1, _H), lambda i: (0, 0)),
        ],
        out_specs=pl.BlockSpec((2, _BLK, 128), lambda i: (0, i, 0)),
        out_shape=jax.ShapeDtypeStruct((2, n, 128), jnp.float32),
    )(t, stats, gamma, beta)


def _tc_pool_head(hs, batch3, l1w, l1b, l2w, l2b, n, g, ncls):
    nblk = n // _BLK

    def body(hs_ref, b_ref, l1w_ref, l1b_ref, l2w_ref, l2b_ref, out_ref,
             pool_acc, cnt_acc):
        i = pl.program_id(0)

        @pl.when(i == 0)
        def _():
            pool_acc[...] = jnp.zeros_like(pool_acc)
            cnt_acc[...] = jnp.zeros_like(cnt_acc)

        bids = b_ref[0, 0, :]
        giota = lax.broadcasted_iota(jnp.int32, (g, _BLK), 0)
        onehot = (giota == bids[None, :]).astype(jnp.float32)
        pool_acc[:, :128] += jnp.dot(onehot, hs_ref[0],
                                     preferred_element_type=jnp.float32)
        pool_acc[:, 128:] += jnp.dot(onehot, hs_ref[1],
                                     preferred_element_type=jnp.float32)
        cnt_acc[...] += jnp.sum(onehot, axis=1, keepdims=True)

        @pl.when(i == nblk - 1)
        def _():
            pooled = pool_acc[...] / jnp.maximum(cnt_acc[...], 1.0)
            o = jnp.dot(pooled, l1w_ref[...],
                        preferred_element_type=jnp.float32) + l1b_ref[...]
            o = jnp.maximum(o, 0.0)
            o = jnp.dot(o, l2w_ref[...],
                        preferred_element_type=jnp.float32) + l2b_ref[...]
            m = jnp.max(o, axis=1, keepdims=True)
            ls = jnp.log(jnp.sum(jnp.exp(o - m), axis=1, keepdims=True))
            out_ref[...] = (o - m) - ls

    return pl.pallas_call(
        body,
        grid=(nblk,),
        in_specs=[
            pl.BlockSpec((2, _BLK, 128), lambda i: (0, i, 0)),
            pl.BlockSpec((1, 1, _BLK), lambda i: (i, 0, 0)),
            pl.BlockSpec((_H, _H), lambda i: (0, 0)),
            pl.BlockSpec((1, _H), lambda i: (0, 0)),
            pl.BlockSpec((_H, ncls), lambda i: (0, 0)),
            pl.BlockSpec((1, ncls), lambda i: (0, 0)),
        ],
        out_specs=pl.BlockSpec((g, ncls), lambda i: (0, 0)),
        out_shape=jax.ShapeDtypeStruct((g, ncls), jnp.float32),
        scratch_shapes=[
            pltpu.VMEM((g, _H), jnp.float32),
            pltpu.VMEM((g, 1), jnp.float32),
        ],
    )(hs, batch3, l1w, l1b, l2w, l2b)


def kernel(x, edge_index, batch, params):
    n, d = x.shape
    e = edge_index.shape[1]
    src = edge_index[0]
    dst = edge_index[1]

    w = d // 2
    hs = jnp.stack([x[:, :w], x[:, w:]], axis=0)

    for p in params["layers"]:
        aggs_flat = _sc_segment_add(hs.reshape(2 * n, w), src, dst, n, w, e)
        aggs = aggs_flat.reshape(2, n, w)
        e_row = jnp.full((1, w), 1.0, jnp.float32) + p["eps"]
        w1a = p["W1"][:w]
        w1b = p["W1"][w:]
        t, stats = _tc_mlp_stats(hs, aggs, e_row, w1a, w1b,
                                 p["b1"].reshape(1, _H), p["W2"],
                                 p["b2"].reshape(1, _H), n, w)
        hs = _tc_bn_relu(t, stats, p["gamma"].reshape(1, _H),
                         p["beta"].reshape(1, _H), n)
        w = 128

    ncls = params["lin2_W"].shape[1]
    batch3 = batch.reshape(n // _BLK, 1, _BLK)
    return _tc_pool_head(hs, batch3, params["lin1_W"],
                         params["lin1_b"].reshape(1, _H), params["lin2_W"],
                         params["lin2_b"].reshape(1, ncls), n, _G,
                         ncls)


# speedup vs baseline: 3.4966x; 3.4966x over previous
import functools

import jax
import jax.numpy as jnp
from jax import lax
from jax.experimental import pallas as pl
from jax.experimental.pallas import tpu as pltpu
from jax.experimental.pallas import tpu_sc as plsc

_NS = 16
_H = 256
_BLK = 1000
_G = 128


def _sc_segment_add(table, src, dst, n, w, e, split_edges):
    tpe = e // (2 * _NS) if split_edges else e // _NS
    b = 80
    nb = tpe // b
    rpt = 632
    zr = 8
    nz_full = rpt // zr

    mesh = plsc.VectorSubcoreMesh(core_axis_name="c", subcore_axis_name="s")

    @functools.partial(
        pl.kernel,
        out_type=jax.ShapeDtypeStruct((2 * n, w), jnp.float32),
        mesh=mesh,
        scratch_types=[
            pltpu.VMEM((b,), jnp.int32),
            pltpu.VMEM((b,), jnp.int32),
            pltpu.VMEM((b, w), jnp.float32),
            pltpu.VMEM((zr, w), jnp.float32),
            pltpu.VMEM_SHARED((n, w), jnp.float32),
            pltpu.SemaphoreType.DMA,
        ],
    )
    def k(table_hbm, src_hbm, dst_hbm, out_hbm, gidx_v, dst_v, rows_v, zbuf,
          acc, sem):
        c = lax.axis_index("c")
        s = lax.axis_index("s")
        zero16 = jnp.zeros((16,), jnp.float32)
        for r in range(zr):
            for q in range(w // 16):
                zbuf[r, pl.ds(q * 16, 16)] = zero16
        rbase = (s * (n // _NS)) // 8 * 8

        def zloop(j, carry):
            pltpu.sync_copy(zbuf, acc.at[pl.ds(rbase + j * zr, zr)])
            return carry

        lax.fori_loop(0, nz_full, zloop, 0)
        plsc.subcore_barrier()

        ebase = (c * _NS + s) * tpe if split_edges else s * tpe
        coff = c * n

        def eloop(i, carry):
            off = ebase + i * b
            pltpu.sync_copy(src_hbm.at[pl.ds(off, b)], gidx_v)
            pltpu.sync_copy(dst_hbm.at[pl.ds(off, b)], dst_v)
            if not split_edges:
                for q in range(b // 16):
                    sl = pl.ds(q * 16, 16)
                    gidx_v[sl] = gidx_v[sl] + coff
            pltpu.async_copy(table_hbm.at[gidx_v], rows_v, sem).wait()
            pltpu.sync_copy(rows_v, acc.at[dst_v], add=True)
            return carry

        lax.fori_loop(0, nb, eloop, 0)
        plsc.subcore_barrier()
        pltpu.sync_copy(acc.at[pl.ds(rbase, rpt)],
                        out_hbm.at[pl.ds(coff + rbase, rpt)])

    return k(table, src, dst)


def _tc_mlp_stats(hs, aggs, e_row, w1a, w1b, b1, w2, b2, n, w, partial):
    nblk = n // _BLK

    def body(hs_ref, agg_ref, e_ref, w1a_ref, w1b_ref, b1_ref, w2_ref, b2_ref,
             t_ref, stats_ref):
        if partial:
            zs = agg_ref[0] + agg_ref[1]
            z0 = hs_ref[0] * e_ref[...] + zs[:, :w]
            z1 = hs_ref[1] * e_ref[...] + zs[:, w:]
        else:
            z0 = hs_ref[0] * e_ref[...] + agg_ref[0]
            z1 = hs_ref[1] * e_ref[...] + agg_ref[1]
        u = jnp.dot(z0, w1a_ref[...], preferred_element_type=jnp.float32)
        u = u + jnp.dot(z1, w1b_ref[...], preferred_element_type=jnp.float32)
        a = jnp.maximum(u + b1_ref[...], 0.0)
        t = jnp.dot(a, w2_ref[...], preferred_element_type=jnp.float32)
        t = t + b2_ref[...]
        t_ref[...] = t

        @pl.when(pl.program_id(0) == 0)
        def _():
            stats_ref[...] = jnp.zeros_like(stats_ref)

        stats_ref[0:1, :] += jnp.sum(t, axis=0, keepdims=True)
        stats_ref[1:2, :] += jnp.sum(t * t, axis=0, keepdims=True)

    return pl.pallas_call(
        body,
        grid=(nblk,),
        in_specs=[
            pl.BlockSpec((2, _BLK, w), lambda i: (0, i, 0)),
            pl.BlockSpec((2, _BLK, 128), lambda i: (0, i, 0)),
            pl.BlockSpec((1, w), lambda i: (0, 0)),
            pl.BlockSpec((w, _H), lambda i: (0, 0)),
            pl.BlockSpec((w, _H), lambda i: (0, 0)),
            pl.BlockSpec((1, _H), lambda i: (0, 0)),
            pl.BlockSpec((_H, _H), lambda i: (0, 0)),
            pl.BlockSpec((1, _H), lambda i: (0, 0)),
        ],
        out_specs=[
            pl.BlockSpec((_BLK, _H), lambda i: (i, 0)),
            pl.BlockSpec((2, _H), lambda i: (0, 0)),
        ],
        out_shape=[
            jax.ShapeDtypeStruct((n, _H), jnp.float32),
            jax.ShapeDtypeStruct((2, _H), jnp.float32),
        ],
    )(hs, aggs, e_row, w1a, w1b, b1, w2, b2)


def _tc_bn_relu(t, stats, gamma, beta, n):
    nblk = n // _BLK
    inv_n = 1.0 / n

    def body(t_ref, stats_ref, gamma_ref, beta_ref, out_ref):
        mu = stats_ref[0:1, :] * inv_n
        var = stats_ref[1:2, :] * inv_n - mu * mu
        inv = lax.rsqrt(var + 1e-5)
        scale = gamma_ref[...] * inv
        shift = beta_ref[...] - mu * scale
        hnew = jnp.maximum(t_ref[...] * scale + shift, 0.0)
        out_ref[0] = hnew[:, :128]
        out_ref[1] = hnew[:, 128:]

    return pl.pallas_call(
        body,
        grid=(nblk,),
        in_specs=[
            pl.BlockSpec((_BLK, _H), lambda i: (i, 0)),
            pl.BlockSpec((2, _H), lambda i: (0, 0)),
            pl.BlockSpec((1, _H), lambda i: (0, 0)),
            pl.BlockSpec((1, _H), lambda i: (0, 0)),
        ],
        out_specs=pl.BlockSpec((2, _BLK, 128), lambda i: (0, i, 0)),
        out_shape=jax.ShapeDtypeStruct((2, n, 128), jnp.float32),
    )(t, stats, gamma, beta)


def _tc_pool_head(hs, batch3, l1w, l1b, l2w, l2b, n, g, ncls):
    nblk = n // _BLK

    def body(hs_ref, b_ref, l1w_ref, l1b_ref, l2w_ref, l2b_ref, out_ref,
             pool_acc, cnt_acc):
        i = pl.program_id(0)

        @pl.when(i == 0)
        def _():
            pool_acc[...] = jnp.zeros_like(pool_acc)
            cnt_acc[...] = jnp.zeros_like(cnt_acc)

        bids = b_ref[0, 0, :]
        giota = lax.broadcasted_iota(jnp.int32, (g, _BLK), 0)
        onehot = (giota == bids[None, :]).astype(jnp.float32)
        pool_acc[:, :128] += jnp.dot(onehot, hs_ref[0],
                                     preferred_element_type=jnp.float32)
        pool_acc[:, 128:] += jnp.dot(onehot, hs_ref[1],
                                     preferred_element_type=jnp.float32)
        cnt_acc[...] += jnp.sum(onehot, axis=1, keepdims=True)

        @pl.when(i == nblk - 1)
        def _():
            pooled = pool_acc[...] / jnp.maximum(cnt_acc[...], 1.0)
            o = jnp.dot(pooled, l1w_ref[...],
                        preferred_element_type=jnp.float32) + l1b_ref[...]
            o = jnp.maximum(o, 0.0)
            o = jnp.dot(o, l2w_ref[...],
                        preferred_element_type=jnp.float32) + l2b_ref[...]
            m = jnp.max(o, axis=1, keepdims=True)
            ls = jnp.log(jnp.sum(jnp.exp(o - m), axis=1, keepdims=True))
            out_ref[...] = (o - m) - ls

    return pl.pallas_call(
        body,
        grid=(nblk,),
        in_specs=[
            pl.BlockSpec((2, _BLK, 128), lambda i: (0, i, 0)),
            pl.BlockSpec((1, 1, _BLK), lambda i: (i, 0, 0)),
            pl.BlockSpec((_H, _H), lambda i: (0, 0)),
            pl.BlockSpec((1, _H), lambda i: (0, 0)),
            pl.BlockSpec((_H, ncls), lambda i: (0, 0)),
            pl.BlockSpec((1, ncls), lambda i: (0, 0)),
        ],
        out_specs=pl.BlockSpec((g, ncls), lambda i: (0, 0)),
        out_shape=jax.ShapeDtypeStruct((g, ncls), jnp.float32),
        scratch_shapes=[
            pltpu.VMEM((g, _H), jnp.float32),
            pltpu.VMEM((g, 1), jnp.float32),
        ],
    )(hs, batch3, l1w, l1b, l2w, l2b)


def kernel(x, edge_index, batch, params):
    n, d = x.shape
    e = edge_index.shape[1]
    src = edge_index[0]
    dst = edge_index[1]

    w = d // 2
    hs = jnp.stack([x[:, :w], x[:, w:]], axis=0)

    for li, p in enumerate(params["layers"]):
        if li == 0:
            aggs_flat = _sc_segment_add(x, src, dst, n, d, e, True)
        else:
            aggs_flat = _sc_segment_add(hs.reshape(2 * n, w), src, dst, n, w,
                                        e, False)
        aggs = aggs_flat.reshape(2, n, aggs_flat.shape[1])
        e_row = jnp.full((1, w), 1.0, jnp.float32) + p["eps"]
        w1a = p["W1"][:w]
        w1b = p["W1"][w:]
        t, stats = _tc_mlp_stats(hs, aggs, e_row, w1a, w1b,
                                 p["b1"].reshape(1, _H), p["W2"],
                                 p["b2"].reshape(1, _H), n, w, li == 0)
        hs = _tc_bn_relu(t, stats, p["gamma"].reshape(1, _H),
                         p["beta"].reshape(1, _H), n)
        w = 128

    ncls = params["lin2_W"].shape[1]
    batch3 = batch.reshape(n // _BLK, 1, _BLK)
    return _tc_pool_head(hs, batch3, params["lin1_W"],
                         params["lin1_b"].reshape(1, _H), params["lin2_W"],
                         params["lin2_b"].reshape(1, ncls), n, _G,
                         ncls)


# speedup vs baseline: 6.5725x; 1.8797x over previous
import functools

import jax
import jax.numpy as jnp
from jax import lax
from jax.experimental import pallas as pl
from jax.experimental.pallas import tpu as pltpu
from jax.experimental.pallas import tpu_sc as plsc

_NS = 16
_H = 256
_BLK = 1000
_G = 128


def _sc_segment_add(table, gsrc, dst, n, w, e, split_edges):
    tpe = e // (2 * _NS) if split_edges else e // _NS
    b = 80
    nb = tpe // b
    rpt = 632
    zr = 8
    nz_full = rpt // zr

    mesh = plsc.VectorSubcoreMesh(core_axis_name="c", subcore_axis_name="s")

    @functools.partial(
        pl.kernel,
        out_type=jax.ShapeDtypeStruct((2 * n, w), jnp.float32),
        mesh=mesh,
        scratch_types=[
            pltpu.VMEM((4, b), jnp.int32),
            pltpu.VMEM((4, b), jnp.int32),
            pltpu.VMEM((2, b, w), jnp.float32),
            pltpu.VMEM((zr, w), jnp.float32),
            pltpu.VMEM_SHARED((n, w), jnp.float32),
            pltpu.SemaphoreType.DMA,
            pltpu.SemaphoreType.DMA,
            pltpu.SemaphoreType.DMA,
        ],
    )
    def k(table_hbm, gsrc_hbm, dst_hbm, out_hbm, idx_v, dst_v, rows_v, zbuf,
          acc, idx_sem, gat_sem, scat_sem):
        c = lax.axis_index("c")
        s = lax.axis_index("s")
        zero16 = jnp.zeros((16,), jnp.float32)
        for r in range(zr):
            for q in range(w // 16):
                zbuf[r, pl.ds(q * 16, 16)] = zero16
        rbase = (s * (n // _NS)) // 8 * 8

        def zloop(j, carry):
            pltpu.sync_copy(zbuf, acc.at[pl.ds(rbase + j * zr, zr)])
            return carry

        lax.fori_loop(0, nz_full, zloop, 0)
        plsc.subcore_barrier()

        if split_edges:
            goff = (c * _NS + s) * tpe
        else:
            goff = c * e + s * tpe
        doff = (c * _NS + s) * tpe if split_edges else s * tpe

        def start_idx(blk, slot):
            pltpu.async_copy(gsrc_hbm.at[pl.ds(goff + blk * b, b)],
                             idx_v.at[slot], idx_sem)
            pltpu.async_copy(dst_hbm.at[pl.ds(doff + blk * b, b)],
                             dst_v.at[slot], idx_sem)

        def drain_idx(slot):
            pltpu.make_async_copy(gsrc_hbm.at[pl.ds(0, b)], idx_v.at[slot],
                                  idx_sem).wait()
            pltpu.make_async_copy(dst_hbm.at[pl.ds(0, b)], dst_v.at[slot],
                                  idx_sem).wait()

        def drain_scat(slot):
            pltpu.make_async_copy(rows_v.at[slot], acc.at[pl.ds(0, b)],
                                  scat_sem).wait()

        start_idx(0, 0)
        start_idx(1, 1)

        def eloop(i, carry):
            q = i % 4
            p = i % 2
            drain_idx(q)

            @pl.when(i >= 2)
            def _():
                drain_scat(p)

            @pl.when(i + 2 < nb)
            def _():
                start_idx(i + 2, (i + 2) % 4)

            pltpu.async_copy(table_hbm.at[idx_v.at[q]], rows_v.at[p],
                             gat_sem).wait()
            pltpu.async_copy(rows_v.at[p], acc.at[dst_v.at[q]], scat_sem,
                             add=True)
            return carry

        lax.fori_loop(0, nb, eloop, 0)
        drain_scat(nb % 2)
        drain_scat((nb + 1) % 2)
        plsc.subcore_barrier()
        pltpu.sync_copy(acc.at[pl.ds(rbase, rpt)],
                        out_hbm.at[pl.ds(c * n + rbase, rpt)])

    return k(table, gsrc, dst)


def _tc_mlp_stats(hs, aggs, e_row, w1a, w1b, b1, w2, b2, n, w, partial):
    nblk = n // _BLK

    def body(hs_ref, agg_ref, e_ref, w1a_ref, w1b_ref, b1_ref, w2_ref, b2_ref,
             t_ref, stats_ref):
        if partial:
            zs = agg_ref[0] + agg_ref[1]
            z0 = hs_ref[0] * e_ref[...] + zs[:, :w]
            z1 = hs_ref[1] * e_ref[...] + zs[:, w:]
        else:
            z0 = hs_ref[0] * e_ref[...] + agg_ref[0]
            z1 = hs_ref[1] * e_ref[...] + agg_ref[1]
        u = jnp.dot(z0, w1a_ref[...], preferred_element_type=jnp.float32)
        u = u + jnp.dot(z1, w1b_ref[...], preferred_element_type=jnp.float32)
        a = jnp.maximum(u + b1_ref[...], 0.0)
        t = jnp.dot(a, w2_ref[...], preferred_element_type=jnp.float32)
        t = t + b2_ref[...]
        t_ref[...] = t

        @pl.when(pl.program_id(0) == 0)
        def _():
            stats_ref[...] = jnp.zeros_like(stats_ref)

        stats_ref[0:1, :] += jnp.sum(t, axis=0, keepdims=True)
        stats_ref[1:2, :] += jnp.sum(t * t, axis=0, keepdims=True)

    return pl.pallas_call(
        body,
        grid=(nblk,),
        in_specs=[
            pl.BlockSpec((2, _BLK, w), lambda i: (0, i, 0)),
            pl.BlockSpec((2, _BLK, 128), lambda i: (0, i, 0)),
            pl.BlockSpec((1, w), lambda i: (0, 0)),
            pl.BlockSpec((w, _H), lambda i: (0, 0)),
            pl.BlockSpec((w, _H), lambda i: (0, 0)),
            pl.BlockSpec((1, _H), lambda i: (0, 0)),
            pl.BlockSpec((_H, _H), lambda i: (0, 0)),
            pl.BlockSpec((1, _H), lambda i: (0, 0)),
        ],
        out_specs=[
            pl.BlockSpec((_BLK, _H), lambda i: (i, 0)),
            pl.BlockSpec((2, _H), lambda i: (0, 0)),
        ],
        out_shape=[
            jax.ShapeDtypeStruct((n, _H), jnp.float32),
            jax.ShapeDtypeStruct((2, _H), jnp.float32),
        ],
    )(hs, aggs, e_row, w1a, w1b, b1, w2, b2)


def _tc_bn_relu(t, stats, gamma, beta, n):
    nblk = n // _BLK
    inv_n = 1.0 / n

    def body(t_ref, stats_ref, gamma_ref, beta_ref, out_ref):
        mu = stats_ref[0:1, :] * inv_n
        var = stats_ref[1:2, :] * inv_n - mu * mu
        inv = lax.rsqrt(var + 1e-5)
        scale = gamma_ref[...] * inv
        shift = beta_ref[...] - mu * scale
        hnew = jnp.maximum(t_ref[...] * scale + shift, 0.0)
        out_ref[0] = hnew[:, :128]
        out_ref[1] = hnew[:, 128:]

    return pl.pallas_call(
        body,
        grid=(nblk,),
        in_specs=[
            pl.BlockSpec((_BLK, _H), lambda i: (i, 0)),
            pl.BlockSpec((2, _H), lambda i: (0, 0)),
            pl.BlockSpec((1, _H), lambda i: (0, 0)),
            pl.BlockSpec((1, _H), lambda i: (0, 0)),
        ],
        out_specs=pl.BlockSpec((2, _BLK, 128), lambda i: (0, i, 0)),
        out_shape=jax.ShapeDtypeStruct((2, n, 128), jnp.float32),
    )(t, stats, gamma, beta)


def _tc_pool_head(hs, batch3, l1w, l1b, l2w, l2b, n, g, ncls):
    nblk = n // _BLK

    def body(hs_ref, b_ref, l1w_ref, l1b_ref, l2w_ref, l2b_ref, out_ref,
             pool_acc, cnt_acc):
        i = pl.program_id(0)

        @pl.when(i == 0)
        def _():
            pool_acc[...] = jnp.zeros_like(pool_acc)
            cnt_acc[...] = jnp.zeros_like(cnt_acc)

        bids = b_ref[0, 0, :]
        giota = lax.broadcasted_iota(jnp.int32, (g, _BLK), 0)
        onehot = (giota == bids[None, :]).astype(jnp.float32)
        pool_acc[:, :128] += jnp.dot(onehot, hs_ref[0],
                                     preferred_element_type=jnp.float32)
        pool_acc[:, 128:] += jnp.dot(onehot, hs_ref[1],
                                     preferred_element_type=jnp.float32)
        cnt_acc[...] += jnp.sum(onehot, axis=1, keepdims=True)

        @pl.when(i == nblk - 1)
        def _():
            pooled = pool_acc[...] / jnp.maximum(cnt_acc[...], 1.0)
            o = jnp.dot(pooled, l1w_ref[...],
                        preferred_element_type=jnp.float32) + l1b_ref[...]
            o = jnp.maximum(o, 0.0)
            o = jnp.dot(o, l2w_ref[...],
                        preferred_element_type=jnp.float32) + l2b_ref[...]
            m = jnp.max(o, axis=1, keepdims=True)
            ls = jnp.log(jnp.sum(jnp.exp(o - m), axis=1, keepdims=True))
            out_ref[...] = (o - m) - ls

    return pl.pallas_call(
        body,
        grid=(nblk,),
        in_specs=[
            pl.BlockSpec((2, _BLK, 128), lambda i: (0, i, 0)),
            pl.BlockSpec((1, 1, _BLK), lambda i: (i, 0, 0)),
            pl.BlockSpec((_H, _H), lambda i: (0, 0)),
            pl.BlockSpec((1, _H), lambda i: (0, 0)),
            pl.BlockSpec((_H, ncls), lambda i: (0, 0)),
            pl.BlockSpec((1, ncls), lambda i: (0, 0)),
        ],
        out_specs=pl.BlockSpec((g, ncls), lambda i: (0, 0)),
        out_shape=jax.ShapeDtypeStruct((g, ncls), jnp.float32),
        scratch_shapes=[
            pltpu.VMEM((g, _H), jnp.float32),
            pltpu.VMEM((g, 1), jnp.float32),
        ],
    )(hs, batch3, l1w, l1b, l2w, l2b)


def kernel(x, edge_index, batch, params):
    n, d = x.shape
    e = edge_index.shape[1]
    src = edge_index[0]
    dst = edge_index[1]

    w = d // 2
    hs = jnp.stack([x[:, :w], x[:, w:]], axis=0)

    gsrc_chunk = jnp.concatenate([src, src + n])

    for li, p in enumerate(params["layers"]):
        if li == 0:
            aggs_flat = _sc_segment_add(x, src, dst, n, d, e, True)
        else:
            aggs_flat = _sc_segment_add(hs.reshape(2 * n, w), gsrc_chunk, dst,
                                        n, w, e, False)
        aggs = aggs_flat.reshape(2, n, aggs_flat.shape[1])
        e_row = jnp.full((1, w), 1.0, jnp.float32) + p["eps"]
        w1a = p["W1"][:w]
        w1b = p["W1"][w:]
        t, stats = _tc_mlp_stats(hs, aggs, e_row, w1a, w1b,
                                 p["b1"].reshape(1, _H), p["W2"],
                                 p["b2"].reshape(1, _H), n, w, li == 0)
        hs = _tc_bn_relu(t, stats, p["gamma"].reshape(1, _H),
                         p["beta"].reshape(1, _H), n)
        w = 128

    ncls = params["lin2_W"].shape[1]
    batch3 = batch.reshape(n // _BLK, 1, _BLK)
    return _tc_pool_head(hs, batch3, params["lin1_W"],
                         params["lin1_b"].reshape(1, _H), params["lin2_W"],
                         params["lin2_b"].reshape(1, ncls), n, _G,
                         ncls)


# speedup vs baseline: 9.4875x; 1.4435x over previous
import functools

import jax
import jax.numpy as jnp
from jax import lax
from jax.experimental import pallas as pl
from jax.experimental.pallas import tpu as pltpu
from jax.experimental.pallas import tpu_sc as plsc

_NS = 16
_H = 256
_BLK = 1000
_G = 128


def _sc_segment_add(table, gsrc, dst, n, w, e, split_edges):
    tpe = e // (2 * _NS) if split_edges else e // _NS
    b = 80
    nb = tpe // b
    rpt = 632
    zr = 8
    nz_full = rpt // zr

    mesh = plsc.VectorSubcoreMesh(core_axis_name="c", subcore_axis_name="s")

    @functools.partial(
        pl.kernel,
        out_type=jax.ShapeDtypeStruct((2 * n, w), jnp.float32),
        mesh=mesh,
        scratch_types=[
            pltpu.VMEM((6, b), jnp.int32),
            pltpu.VMEM((6, b), jnp.int32),
            pltpu.VMEM((4, b, w), jnp.float32),
            pltpu.VMEM((zr, w), jnp.float32),
            pltpu.VMEM_SHARED((n, w), jnp.float32),
            pltpu.SemaphoreType.DMA,
            pltpu.SemaphoreType.DMA,
            pltpu.SemaphoreType.DMA,
        ],
    )
    def k(table_hbm, gsrc_hbm, dst_hbm, out_hbm, idx_v, dst_v, rows_v, zbuf,
          acc, idx_sem, gat_sem, scat_sem):
        c = lax.axis_index("c")
        s = lax.axis_index("s")
        zero16 = jnp.zeros((16,), jnp.float32)
        for r in range(zr):
            for q in range(w // 16):
                zbuf[r, pl.ds(q * 16, 16)] = zero16
        rbase = (s * (n // _NS)) // 8 * 8

        def zloop(j, carry):
            pltpu.sync_copy(zbuf, acc.at[pl.ds(rbase + j * zr, zr)])
            return carry

        lax.fori_loop(0, nz_full, zloop, 0)
        plsc.subcore_barrier()

        if split_edges:
            goff = (c * _NS + s) * tpe
        else:
            goff = c * e + s * tpe
        doff = (c * _NS + s) * tpe if split_edges else s * tpe

        def start_idx(blk, slot):
            pltpu.async_copy(gsrc_hbm.at[pl.ds(goff + blk * b, b)],
                             idx_v.at[slot], idx_sem)
            pltpu.async_copy(dst_hbm.at[pl.ds(doff + blk * b, b)],
                             dst_v.at[slot], idx_sem)

        def drain_idx(slot):
            pltpu.make_async_copy(gsrc_hbm.at[pl.ds(0, b)], idx_v.at[slot],
                                  idx_sem).wait()
            pltpu.make_async_copy(dst_hbm.at[pl.ds(0, b)], dst_v.at[slot],
                                  idx_sem).wait()

        def drain_scat(slot):
            pltpu.make_async_copy(rows_v.at[slot], acc.at[pl.ds(0, b)],
                                  scat_sem).wait()

        def drain_gat(slot):
            pltpu.make_async_copy(table_hbm.at[pl.ds(0, b)], rows_v.at[slot],
                                  gat_sem).wait()

        def start_gat(blk, islot, rslot):
            pltpu.async_copy(table_hbm.at[idx_v.at[islot]], rows_v.at[rslot],
                             gat_sem)

        for j in range(4):
            start_idx(j, j)
        drain_idx(0)
        start_gat(0, 0, 0)
        drain_idx(1)
        start_gat(1, 1, 1)

        def eloop(i, carry):
            drain_gat(i % 4)

            @pl.when(i >= 2)
            def _():
                drain_scat((i + 2) % 4)

            @pl.when(i + 2 < nb)
            def _():
                drain_idx((i + 2) % 6)
                start_gat(i + 2, (i + 2) % 6, (i + 2) % 4)

            @pl.when(i + 4 < nb)
            def _():
                start_idx(i + 4, (i + 4) % 6)

            pltpu.async_copy(rows_v.at[i % 4], acc.at[dst_v.at[i % 6]],
                             scat_sem, add=True)
            return carry

        lax.fori_loop(0, nb, eloop, 0)
        drain_scat((nb - 2) % 4)
        drain_scat((nb - 1) % 4)
        plsc.subcore_barrier()
        pltpu.sync_copy(acc.at[pl.ds(rbase, rpt)],
                        out_hbm.at[pl.ds(c * n + rbase, rpt)])

    return k(table, gsrc, dst)


def _tc_mlp_stats(hs, aggs, e_row, w1a, w1b, b1, w2, b2, n, w, partial):
    nblk = n // _BLK

    def body(hs_ref, agg_ref, e_ref, w1a_ref, w1b_ref, b1_ref, w2_ref, b2_ref,
             t_ref, stats_ref):
        if partial:
            zs = agg_ref[0] + agg_ref[1]
            z0 = hs_ref[0] * e_ref[...] + zs[:, :w]
            z1 = hs_ref[1] * e_ref[...] + zs[:, w:]
        else:
            z0 = hs_ref[0] * e_ref[...] + agg_ref[0]
            z1 = hs_ref[1] * e_ref[...] + agg_ref[1]
        u = jnp.dot(z0, w1a_ref[...], preferred_element_type=jnp.float32)
        u = u + jnp.dot(z1, w1b_ref[...], preferred_element_type=jnp.float32)
        a = jnp.maximum(u + b1_ref[...], 0.0)
        t = jnp.dot(a, w2_ref[...], preferred_element_type=jnp.float32)
        t = t + b2_ref[...]
        t_ref[...] = t

        @pl.when(pl.program_id(0) == 0)
        def _():
            stats_ref[...] = jnp.zeros_like(stats_ref)

        stats_ref[0:1, :] += jnp.sum(t, axis=0, keepdims=True)
        stats_ref[1:2, :] += jnp.sum(t * t, axis=0, keepdims=True)

    return pl.pallas_call(
        body,
        grid=(nblk,),
        in_specs=[
            pl.BlockSpec((2, _BLK, w), lambda i: (0, i, 0)),
            pl.BlockSpec((2, _BLK, 128), lambda i: (0, i, 0)),
            pl.BlockSpec((1, w), lambda i: (0, 0)),
            pl.BlockSpec((w, _H), lambda i: (0, 0)),
            pl.BlockSpec((w, _H), lambda i: (0, 0)),
            pl.BlockSpec((1, _H), lambda i: (0, 0)),
            pl.BlockSpec((_H, _H), lambda i: (0, 0)),
            pl.BlockSpec((1, _H), lambda i: (0, 0)),
        ],
        out_specs=[
            pl.BlockSpec((_BLK, _H), lambda i: (i, 0)),
            pl.BlockSpec((2, _H), lambda i: (0, 0)),
        ],
        out_shape=[
            jax.ShapeDtypeStruct((n, _H), jnp.float32),
            jax.ShapeDtypeStruct((2, _H), jnp.float32),
        ],
    )(hs, aggs, e_row, w1a, w1b, b1, w2, b2)


def _tc_bn_relu(t, stats, gamma, beta, n):
    nblk = n // _BLK
    inv_n = 1.0 / n

    def body(t_ref, stats_ref, gamma_ref, beta_ref, out_ref):
        mu = stats_ref[0:1, :] * inv_n
        var = stats_ref[1:2, :] * inv_n - mu * mu
        inv = lax.rsqrt(var + 1e-5)
        scale = gamma_ref[...] * inv
        shift = beta_ref[...] - mu * scale
        hnew = jnp.maximum(t_ref[...] * scale + shift, 0.0)
        out_ref[0] = hnew[:, :128]
        out_ref[1] = hnew[:, 128:]

    return pl.pallas_call(
        body,
        grid=(nblk,),
        in_specs=[
            pl.BlockSpec((_BLK, _H), lambda i: (i, 0)),
            pl.BlockSpec((2, _H), lambda i: (0, 0)),
            pl.BlockSpec((1, _H), lambda i: (0, 0)),
            pl.BlockSpec((1, _H), lambda i: (0, 0)),
        ],
        out_specs=pl.BlockSpec((2, _BLK, 128), lambda i: (0, i, 0)),
        out_shape=jax.ShapeDtypeStruct((2, n, 128), jnp.float32),
    )(t, stats, gamma, beta)


def _tc_pool_head(hs, batch3, l1w, l1b, l2w, l2b, n, g, ncls):
    nblk = n // _BLK

    def body(hs_ref, b_ref, l1w_ref, l1b_ref, l2w_ref, l2b_ref, out_ref,
             pool_acc, cnt_acc):
        i = pl.program_id(0)

        @pl.when(i == 0)
        def _():
            pool_acc[...] = jnp.zeros_like(pool_acc)
            cnt_acc[...] = jnp.zeros_like(cnt_acc)

        bids = b_ref[0, 0, :]
        giota = lax.broadcasted_iota(jnp.int32, (g, _BLK), 0)
        onehot = (giota == bids[None, :]).astype(jnp.float32)
        pool_acc[:, :128] += jnp.dot(onehot, hs_ref[0],
                                     preferred_element_type=jnp.float32)
        pool_acc[:, 128:] += jnp.dot(onehot, hs_ref[1],
                                     preferred_element_type=jnp.float32)
        cnt_acc[...] += jnp.sum(onehot, axis=1, keepdims=True)

        @pl.when(i == nblk - 1)
        def _():
            pooled = pool_acc[...] / jnp.maximum(cnt_acc[...], 1.0)
            o = jnp.dot(pooled, l1w_ref[...],
                        preferred_element_type=jnp.float32) + l1b_ref[...]
            o = jnp.maximum(o, 0.0)
            o = jnp.dot(o, l2w_ref[...],
                        preferred_element_type=jnp.float32) + l2b_ref[...]
            m = jnp.max(o, axis=1, keepdims=True)
            ls = jnp.log(jnp.sum(jnp.exp(o - m), axis=1, keepdims=True))
            out_ref[...] = (o - m) - ls

    return pl.pallas_call(
        body,
        grid=(nblk,),
        in_specs=[
            pl.BlockSpec((2, _BLK, 128), lambda i: (0, i, 0)),
            pl.BlockSpec((1, 1, _BLK), lambda i: (i, 0, 0)),
            pl.BlockSpec((_H, _H), lambda i: (0, 0)),
            pl.BlockSpec((1, _H), lambda i: (0, 0)),
            pl.BlockSpec((_H, ncls), lambda i: (0, 0)),
            pl.BlockSpec((1, ncls), lambda i: (0, 0)),
        ],
        out_specs=pl.BlockSpec((g, ncls), lambda i: (0, 0)),
        out_shape=jax.ShapeDtypeStruct((g, ncls), jnp.float32),
        scratch_shapes=[
            pltpu.VMEM((g, _H), jnp.float32),
            pltpu.VMEM((g, 1), jnp.float32),
        ],
    )(hs, batch3, l1w, l1b, l2w, l2b)


def kernel(x, edge_index, batch, params):
    n, d = x.shape
    e = edge_index.shape[1]
    src = edge_index[0]
    dst = edge_index[1]

    w = d // 2
    hs = jnp.stack([x[:, :w], x[:, w:]], axis=0)

    gsrc_chunk = jnp.concatenate([src, src + n])

    for li, p in enumerate(params["layers"]):
        if li == 0:
            aggs_flat = _sc_segment_add(x, src, dst, n, d, e, True)
        else:
            aggs_flat = _sc_segment_add(hs.reshape(2 * n, w), gsrc_chunk, dst,
                                        n, w, e, False)
        aggs = aggs_flat.reshape(2, n, aggs_flat.shape[1])
        e_row = jnp.full((1, w), 1.0, jnp.float32) + p["eps"]
        w1a = p["W1"][:w]
        w1b = p["W1"][w:]
        t, stats = _tc_mlp_stats(hs, aggs, e_row, w1a, w1b,
                                 p["b1"].reshape(1, _H), p["W2"],
                                 p["b2"].reshape(1, _H), n, w, li == 0)
        hs = _tc_bn_relu(t, stats, p["gamma"].reshape(1, _H),
                         p["beta"].reshape(1, _H), n)
        w = 128

    ncls = params["lin2_W"].shape[1]
    batch3 = batch.reshape(n // _BLK, 1, _BLK)
    return _tc_pool_head(hs, batch3, params["lin1_W"],
                         params["lin1_b"].reshape(1, _H), params["lin2_W"],
                         params["lin2_b"].reshape(1, ncls), n, _G,
                         ncls)


# speedup vs baseline: 9.4901x; 1.0003x over previous
import functools

import jax
import jax.numpy as jnp
from jax import lax
from jax.experimental import pallas as pl
from jax.experimental.pallas import tpu as pltpu
from jax.experimental.pallas import tpu_sc as plsc

_NS = 16
_H = 256
_BLK = 1000
_G = 128


def _sc_segment_add(table, gsrc, dst, n, w, e, split_edges):
    tpe = e // (2 * _NS) if split_edges else e // _NS
    b = 80
    nb = tpe // b
    rpt = 632
    zr = 8
    nz_full = rpt // zr

    mesh = plsc.VectorSubcoreMesh(core_axis_name="c", subcore_axis_name="s")

    @functools.partial(
        pl.kernel,
        out_type=jax.ShapeDtypeStruct((2 * n, w), jnp.float32),
        mesh=mesh,
        scratch_types=[
            pltpu.VMEM((6, b), jnp.int32),
            pltpu.VMEM((6, b), jnp.int32),
            pltpu.VMEM((4, b, w), jnp.float32),
            pltpu.VMEM((zr, w), jnp.float32),
            pltpu.VMEM_SHARED((n, w), jnp.float32),
            pltpu.SemaphoreType.DMA,
            pltpu.SemaphoreType.DMA,
            pltpu.SemaphoreType.DMA,
        ],
    )
    def k(table_hbm, gsrc_hbm, dst_hbm, out_hbm, idx_v, dst_v, rows_v, zbuf,
          acc, idx_sem, gat_sem, scat_sem):
        c = lax.axis_index("c")
        s = lax.axis_index("s")
        zero16 = jnp.zeros((16,), jnp.float32)
        for r in range(zr):
            for q in range(w // 16):
                zbuf[r, pl.ds(q * 16, 16)] = zero16
        rbase = (s * (n // _NS)) // 8 * 8

        def zloop(j, carry):
            pltpu.sync_copy(zbuf, acc.at[pl.ds(rbase + j * zr, zr)])
            return carry

        lax.fori_loop(0, nz_full, zloop, 0)
        plsc.subcore_barrier()

        if split_edges:
            goff = (c * _NS + s) * tpe
        else:
            goff = c * e + s * tpe
        doff = (c * _NS + s) * tpe if split_edges else s * tpe

        def start_idx(blk, slot):
            pltpu.async_copy(gsrc_hbm.at[pl.ds(goff + blk * b, b)],
                             idx_v.at[slot], idx_sem)
            pltpu.async_copy(dst_hbm.at[pl.ds(doff + blk * b, b)],
                             dst_v.at[slot], idx_sem)

        def drain_idx(slot):
            pltpu.make_async_copy(gsrc_hbm.at[pl.ds(0, b)], idx_v.at[slot],
                                  idx_sem).wait()
            pltpu.make_async_copy(dst_hbm.at[pl.ds(0, b)], dst_v.at[slot],
                                  idx_sem).wait()

        def drain_scat(slot):
            pltpu.make_async_copy(rows_v.at[slot], acc.at[pl.ds(0, b)],
                                  scat_sem).wait()

        def drain_gat(slot):
            pltpu.make_async_copy(table_hbm.at[pl.ds(0, b)], rows_v.at[slot],
                                  gat_sem).wait()

        def start_gat(blk, islot, rslot):
            pltpu.async_copy(table_hbm.at[idx_v.at[islot]], rows_v.at[rslot],
                             gat_sem)

        for j in range(4):
            start_idx(j, j)
        drain_idx(0)
        start_gat(0, 0, 0)
        drain_idx(1)
        start_gat(1, 1, 1)
        nbi = nb // 2

        def eloop(ii, carry):
            i0 = 2 * ii
            r0 = 2 * (ii % 2)
            o0 = 2 - r0
            q0 = 2 * (ii % 3)
            f0 = 2 * ((ii + 1) % 3)
            g0 = 2 * ((ii + 2) % 3)

            drain_gat(r0)

            @pl.when(ii >= 1)
            def _():
                drain_scat(o0)

            @pl.when(i0 + 2 < nb)
            def _():
                drain_idx(f0)
                start_gat(i0 + 2, f0, o0)

            @pl.when(i0 + 4 < nb)
            def _():
                start_idx(i0 + 4, g0)

            pltpu.async_copy(rows_v.at[r0], acc.at[dst_v.at[q0]],
                             scat_sem, add=True)

            drain_gat(r0 + 1)

            @pl.when(ii >= 1)
            def _():
                drain_scat(o0 + 1)

            @pl.when(i0 + 3 < nb)
            def _():
                drain_idx(f0 + 1)
                start_gat(i0 + 3, f0 + 1, o0 + 1)

            @pl.when(i0 + 5 < nb)
            def _():
                start_idx(i0 + 5, g0 + 1)

            pltpu.async_copy(rows_v.at[r0 + 1], acc.at[dst_v.at[q0 + 1]],
                             scat_sem, add=True)
            return carry

        lax.fori_loop(0, nbi, eloop, 0)
        if nb % 2:
            rt = (nb - 1) % 4
            qt = (nb - 1) % 6
            drain_gat(rt)
            drain_scat((nb - 3) % 4)
            pltpu.async_copy(rows_v.at[rt], acc.at[dst_v.at[qt]], scat_sem,
                             add=True)
        drain_scat((nb - 2) % 4)
        drain_scat((nb - 1) % 4)
        plsc.subcore_barrier()
        pltpu.sync_copy(acc.at[pl.ds(rbase, rpt)],
                        out_hbm.at[pl.ds(c * n + rbase, rpt)])

    return k(table, gsrc, dst)


def _tc_mlp_stats(hs, aggs, e_row, w1a, w1b, b1, w2, b2, n, w, partial):
    nblk = n // _BLK

    def body(hs_ref, agg_ref, e_ref, w1a_ref, w1b_ref, b1_ref, w2_ref, b2_ref,
             t_ref, stats_ref):
        if partial:
            zs = agg_ref[0] + agg_ref[1]
            z0 = hs_ref[0] * e_ref[...] + zs[:, :w]
            z1 = hs_ref[1] * e_ref[...] + zs[:, w:]
        else:
            z0 = hs_ref[0] * e_ref[...] + agg_ref[0]
            z1 = hs_ref[1] * e_ref[...] + agg_ref[1]
        u = jnp.dot(z0, w1a_ref[...], preferred_element_type=jnp.float32)
        u = u + jnp.dot(z1, w1b_ref[...], preferred_element_type=jnp.float32)
        a = jnp.maximum(u + b1_ref[...], 0.0)
        t = jnp.dot(a, w2_ref[...], preferred_element_type=jnp.float32)
        t = t + b2_ref[...]
        t_ref[...] = t

        @pl.when(pl.program_id(0) == 0)
        def _():
            stats_ref[...] = jnp.zeros_like(stats_ref)

        stats_ref[0:1, :] += jnp.sum(t, axis=0, keepdims=True)
        stats_ref[1:2, :] += jnp.sum(t * t, axis=0, keepdims=True)

    return pl.pallas_call(
        body,
        grid=(nblk,),
        in_specs=[
            pl.BlockSpec((2, _BLK, w), lambda i: (0, i, 0)),
            pl.BlockSpec((2, _BLK, 128), lambda i: (0, i, 0)),
            pl.BlockSpec((1, w), lambda i: (0, 0)),
            pl.BlockSpec((w, _H), lambda i: (0, 0)),
            pl.BlockSpec((w, _H), lambda i: (0, 0)),
            pl.BlockSpec((1, _H), lambda i: (0, 0)),
            pl.BlockSpec((_H, _H), lambda i: (0, 0)),
            pl.BlockSpec((1, _H), lambda i: (0, 0)),
        ],
        out_specs=[
            pl.BlockSpec((_BLK, _H), lambda i: (i, 0)),
            pl.BlockSpec((2, _H), lambda i: (0, 0)),
        ],
        out_shape=[
            jax.ShapeDtypeStruct((n, _H), jnp.float32),
            jax.ShapeDtypeStruct((2, _H), jnp.float32),
        ],
    )(hs, aggs, e_row, w1a, w1b, b1, w2, b2)


def _tc_bn_relu(t, stats, gamma, beta, n):
    nblk = n // _BLK
    inv_n = 1.0 / n

    def body(t_ref, stats_ref, gamma_ref, beta_ref, out_ref):
        mu = stats_ref[0:1, :] * inv_n
        var = stats_ref[1:2, :] * inv_n - mu * mu
        inv = lax.rsqrt(var + 1e-5)
        scale = gamma_ref[...] * inv
        shift = beta_ref[...] - mu * scale
        hnew = jnp.maximum(t_ref[...] * scale + shift, 0.0)
        out_ref[0] = hnew[:, :128]
        out_ref[1] = hnew[:, 128:]

    return pl.pallas_call(
        body,
        grid=(nblk,),
        in_specs=[
            pl.BlockSpec((_BLK, _H), lambda i: (i, 0)),
            pl.BlockSpec((2, _H), lambda i: (0, 0)),
            pl.BlockSpec((1, _H), lambda i: (0, 0)),
            pl.BlockSpec((1, _H), lambda i: (0, 0)),
        ],
        out_specs=pl.BlockSpec((2, _BLK, 128), lambda i: (0, i, 0)),
        out_shape=jax.ShapeDtypeStruct((2, n, 128), jnp.float32),
    )(t, stats, gamma, beta)


def _tc_pool_head(hs, batch3, l1w, l1b, l2w, l2b, n, g, ncls):
    nblk = n // _BLK

    def body(hs_ref, b_ref, l1w_ref, l1b_ref, l2w_ref, l2b_ref, out_ref,
             pool_acc, cnt_acc):
        i = pl.program_id(0)

        @pl.when(i == 0)
        def _():
            pool_acc[...] = jnp.zeros_like(pool_acc)
            cnt_acc[...] = jnp.zeros_like(cnt_acc)

        bids = b_ref[0, 0, :]
        giota = lax.broadcasted_iota(jnp.int32, (g, _BLK), 0)
        onehot = (giota == bids[None, :]).astype(jnp.float32)
        pool_acc[:, :128] += jnp.dot(onehot, hs_ref[0],
                                     preferred_element_type=jnp.float32)
        pool_acc[:, 128:] += jnp.dot(onehot, hs_ref[1],
                                     preferred_element_type=jnp.float32)
        cnt_acc[...] += jnp.sum(onehot, axis=1, keepdims=True)

        @pl.when(i == nblk - 1)
        def _():
            pooled = pool_acc[...] / jnp.maximum(cnt_acc[...], 1.0)
            o = jnp.dot(pooled, l1w_ref[...],
                        preferred_element_type=jnp.float32) + l1b_ref[...]
            o = jnp.maximum(o, 0.0)
            o = jnp.dot(o, l2w_ref[...],
                        preferred_element_type=jnp.float32) + l2b_ref[...]
            m = jnp.max(o, axis=1, keepdims=True)
            ls = jnp.log(jnp.sum(jnp.exp(o - m), axis=1, keepdims=True))
            out_ref[...] = (o - m) - ls

    return pl.pallas_call(
        body,
        grid=(nblk,),
        in_specs=[
            pl.BlockSpec((2, _BLK, 128), lambda i: (0, i, 0)),
            pl.BlockSpec((1, 1, _BLK), lambda i: (i, 0, 0)),
            pl.BlockSpec((_H, _H), lambda i: (0, 0)),
            pl.BlockSpec((1, _H), lambda i: (0, 0)),
            pl.BlockSpec((_H, ncls), lambda i: (0, 0)),
            pl.BlockSpec((1, ncls), lambda i: (0, 0)),
        ],
        out_specs=pl.BlockSpec((g, ncls), lambda i: (0, 0)),
        out_shape=jax.ShapeDtypeStruct((g, ncls), jnp.float32),
        scratch_shapes=[
            pltpu.VMEM((g, _H), jnp.float32),
            pltpu.VMEM((g, 1), jnp.float32),
        ],
    )(hs, batch3, l1w, l1b, l2w, l2b)


def kernel(x, edge_index, batch, params):
    n, d = x.shape
    e = edge_index.shape[1]
    src = edge_index[0]
    dst = edge_index[1]

    w = d // 2
    hs = jnp.stack([x[:, :w], x[:, w:]], axis=0)

    gsrc_chunk = jnp.concatenate([src, src + n])

    for li, p in enumerate(params["layers"]):
        if li == 0:
            aggs_flat = _sc_segment_add(x, src, dst, n, d, e, True)
        else:
            aggs_flat = _sc_segment_add(hs.reshape(2 * n, w), gsrc_chunk, dst,
                                        n, w, e, False)
        aggs = aggs_flat.reshape(2, n, aggs_flat.shape[1])
        e_row = jnp.full((1, w), 1.0, jnp.float32) + p["eps"]
        w1a = p["W1"][:w]
        w1b = p["W1"][w:]
        t, stats = _tc_mlp_stats(hs, aggs, e_row, w1a, w1b,
                                 p["b1"].reshape(1, _H), p["W2"],
                                 p["b2"].reshape(1, _H), n, w, li == 0)
        hs = _tc_bn_relu(t, stats, p["gamma"].reshape(1, _H),
                         p["beta"].reshape(1, _H), n)
        w = 128

    ncls = params["lin2_W"].shape[1]
    batch3 = batch.reshape(n // _BLK, 1, _BLK)
    return _tc_pool_head(hs, batch3, params["lin1_W"],
                         params["lin1_b"].reshape(1, _H), params["lin2_W"],
                         params["lin2_b"].reshape(1, ncls), n, _G,
                         ncls)


# speedup vs baseline: 9.7294x; 1.0252x over previous
import functools

import jax
import jax.numpy as jnp
from jax import lax
from jax.experimental import pallas as pl
from jax.experimental.pallas import tpu as pltpu
from jax.experimental.pallas import tpu_sc as plsc

_NS = 16
_H = 256
_BLK = 1000
_G = 128


def _sc_segment_add(table, gsrc, dst, n, w, e, split_edges):
    tpe = e // (2 * _NS) if split_edges else e // _NS
    b = 80
    nb = tpe // b
    rpt = 632
    zr = 8
    nz_full = rpt // zr

    mesh = plsc.VectorSubcoreMesh(core_axis_name="c", subcore_axis_name="s")

    @functools.partial(
        pl.kernel,
        out_type=jax.ShapeDtypeStruct((2 * n, w), jnp.float32),
        mesh=mesh,
        scratch_types=[
            pltpu.VMEM((6, b), jnp.int32),
            pltpu.VMEM((6, b), jnp.int32),
            pltpu.VMEM((4, b, w), jnp.float32),
            pltpu.VMEM((zr, w), jnp.float32),
            pltpu.VMEM_SHARED((n, w), jnp.float32),
            pltpu.SemaphoreType.DMA,
            pltpu.SemaphoreType.DMA,
            pltpu.SemaphoreType.DMA,
        ],
    )
    def k(table_hbm, gsrc_hbm, dst_hbm, out_hbm, idx_v, dst_v, rows_v, zbuf,
          acc, idx_sem, gat_sem, scat_sem):
        c = lax.axis_index("c")
        s = lax.axis_index("s")
        zero16 = jnp.zeros((16,), jnp.float32)
        for r in range(zr):
            for q in range(w // 16):
                zbuf[r, pl.ds(q * 16, 16)] = zero16
        rbase = (s * (n // _NS)) // 8 * 8

        def zloop(j, carry):
            pltpu.sync_copy(zbuf, acc.at[pl.ds(rbase + j * zr, zr)])
            return carry

        lax.fori_loop(0, nz_full, zloop, 0)
        plsc.subcore_barrier()

        if split_edges:
            goff = (c * _NS + s) * tpe
        else:
            goff = c * e + s * tpe
        doff = (c * _NS + s) * tpe if split_edges else s * tpe

        def start_idx(blk, slot):
            pltpu.async_copy(gsrc_hbm.at[pl.ds(goff + blk * b, b)],
                             idx_v.at[slot], idx_sem)
            pltpu.async_copy(dst_hbm.at[pl.ds(doff + blk * b, b)],
                             dst_v.at[slot], idx_sem)

        def drain_idx(slot):
            pltpu.make_async_copy(gsrc_hbm.at[pl.ds(0, b)], idx_v.at[slot],
                                  idx_sem).wait()
            pltpu.make_async_copy(dst_hbm.at[pl.ds(0, b)], dst_v.at[slot],
                                  idx_sem).wait()

        def drain_scat(slot):
            pltpu.make_async_copy(rows_v.at[slot], acc.at[pl.ds(0, b)],
                                  scat_sem).wait()

        def drain_gat(slot):
            pltpu.make_async_copy(table_hbm.at[pl.ds(0, b)], rows_v.at[slot],
                                  gat_sem).wait()

        def start_gat(blk, islot, rslot):
            pltpu.async_copy(table_hbm.at[idx_v.at[islot]], rows_v.at[rslot],
                             gat_sem)

        for j in range(4):
            start_idx(j, j)
        drain_idx(0)
        start_gat(0, 0, 0)
        drain_idx(1)
        start_gat(1, 1, 1)
        nbi = nb // 2

        def eloop(ii, carry):
            i0 = 2 * ii
            r0 = 2 * (ii % 2)
            o0 = 2 - r0
            q0 = 2 * (ii % 3)
            f0 = 2 * ((ii + 1) % 3)
            g0 = 2 * ((ii + 2) % 3)

            drain_gat(r0)

            @pl.when(ii >= 1)
            def _():
                drain_scat(o0)

            @pl.when(i0 + 2 < nb)
            def _():
                drain_idx(f0)
                start_gat(i0 + 2, f0, o0)

            @pl.when(i0 + 4 < nb)
            def _():
                start_idx(i0 + 4, g0)

            pltpu.async_copy(rows_v.at[r0], acc.at[dst_v.at[q0]],
                             scat_sem, add=True)

            drain_gat(r0 + 1)

            @pl.when(ii >= 1)
            def _():
                drain_scat(o0 + 1)

            @pl.when(i0 + 3 < nb)
            def _():
                drain_idx(f0 + 1)
                start_gat(i0 + 3, f0 + 1, o0 + 1)

            @pl.when(i0 + 5 < nb)
            def _():
                start_idx(i0 + 5, g0 + 1)

            pltpu.async_copy(rows_v.at[r0 + 1], acc.at[dst_v.at[q0 + 1]],
                             scat_sem, add=True)
            return carry

        lax.fori_loop(0, nbi, eloop, 0)
        if nb % 2:
            rt = (nb - 1) % 4
            qt = (nb - 1) % 6
            drain_gat(rt)
            drain_scat((nb - 3) % 4)
            pltpu.async_copy(rows_v.at[rt], acc.at[dst_v.at[qt]], scat_sem,
                             add=True)
        drain_scat((nb - 2) % 4)
        drain_scat((nb - 1) % 4)
        plsc.subcore_barrier()
        pltpu.sync_copy(acc.at[pl.ds(rbase, rpt)],
                        out_hbm.at[pl.ds(c * n + rbase, rpt)])

    return k(table, gsrc, dst)


def _tc_mlp_stats(hs, aggs, e_row, w1a, w1b, b1, w2, b2, n, w):
    nblk = n // _BLK

    def body(hs_ref, agg_ref, e_ref, w1a_ref, w1b_ref, b1_ref, w2_ref, b2_ref,
             t_ref, stats_ref):
        z0 = hs_ref[0] * e_ref[...] + agg_ref[0]
        z1 = hs_ref[1] * e_ref[...] + agg_ref[1]
        u = jnp.dot(z0, w1a_ref[...], preferred_element_type=jnp.float32)
        u = u + jnp.dot(z1, w1b_ref[...], preferred_element_type=jnp.float32)
        a = jnp.maximum(u + b1_ref[...], 0.0)
        t = jnp.dot(a, w2_ref[...], preferred_element_type=jnp.float32)
        t = t + b2_ref[...]
        t_ref[...] = t

        @pl.when(pl.program_id(0) == 0)
        def _():
            stats_ref[...] = jnp.zeros_like(stats_ref)

        stats_ref[0:1, :] += jnp.sum(t, axis=0, keepdims=True)
        stats_ref[1:2, :] += jnp.sum(t * t, axis=0, keepdims=True)

    return pl.pallas_call(
        body,
        grid=(nblk,),
        in_specs=[
            pl.BlockSpec((2, _BLK, w), lambda i: (0, i, 0)),
            pl.BlockSpec((2, _BLK, 128), lambda i: (0, i, 0)),
            pl.BlockSpec((1, w), lambda i: (0, 0)),
            pl.BlockSpec((w, _H), lambda i: (0, 0)),
            pl.BlockSpec((w, _H), lambda i: (0, 0)),
            pl.BlockSpec((1, _H), lambda i: (0, 0)),
            pl.BlockSpec((_H, _H), lambda i: (0, 0)),
            pl.BlockSpec((1, _H), lambda i: (0, 0)),
        ],
        out_specs=[
            pl.BlockSpec((_BLK, _H), lambda i: (i, 0)),
            pl.BlockSpec((2, _H), lambda i: (0, 0)),
        ],
        out_shape=[
            jax.ShapeDtypeStruct((n, _H), jnp.float32),
            jax.ShapeDtypeStruct((2, _H), jnp.float32),
        ],
    )(hs, aggs, e_row, w1a, w1b, b1, w2, b2)


def _tc_mlp_stats_l1(x, aggs, e_row, w1, b1, w2, b2, n, d):
    nblk = n // _BLK

    def body(x_ref, agg_ref, e_ref, w1_ref, b1_ref, w2_ref, b2_ref,
             t_ref, stats_ref):
        z = x_ref[...] * e_ref[...] + agg_ref[0] + agg_ref[1]
        u = jnp.dot(z, w1_ref[...], preferred_element_type=jnp.float32)
        a = jnp.maximum(u + b1_ref[...], 0.0)
        t = jnp.dot(a, w2_ref[...], preferred_element_type=jnp.float32)
        t = t + b2_ref[...]
        t_ref[...] = t

        @pl.when(pl.program_id(0) == 0)
        def _():
            stats_ref[...] = jnp.zeros_like(stats_ref)

        stats_ref[0:1, :] += jnp.sum(t, axis=0, keepdims=True)
        stats_ref[1:2, :] += jnp.sum(t * t, axis=0, keepdims=True)

    return pl.pallas_call(
        body,
        grid=(nblk,),
        in_specs=[
            pl.BlockSpec((_BLK, d), lambda i: (i, 0)),
            pl.BlockSpec((2, _BLK, d), lambda i: (0, i, 0)),
            pl.BlockSpec((1, d), lambda i: (0, 0)),
            pl.BlockSpec((d, _H), lambda i: (0, 0)),
            pl.BlockSpec((1, _H), lambda i: (0, 0)),
            pl.BlockSpec((_H, _H), lambda i: (0, 0)),
            pl.BlockSpec((1, _H), lambda i: (0, 0)),
        ],
        out_specs=[
            pl.BlockSpec((_BLK, _H), lambda i: (i, 0)),
            pl.BlockSpec((2, _H), lambda i: (0, 0)),
        ],
        out_shape=[
            jax.ShapeDtypeStruct((n, _H), jnp.float32),
            jax.ShapeDtypeStruct((2, _H), jnp.float32),
        ],
    )(x, aggs, e_row, w1, b1, w2, b2)


def _tc_bn_relu(t, stats, gamma, beta, n):
    nblk = n // _BLK
    inv_n = 1.0 / n

    def body(t_ref, stats_ref, gamma_ref, beta_ref, out_ref):
        mu = stats_ref[0:1, :] * inv_n
        var = stats_ref[1:2, :] * inv_n - mu * mu
        inv = lax.rsqrt(var + 1e-5)
        scale = gamma_ref[...] * inv
        shift = beta_ref[...] - mu * scale
        hnew = jnp.maximum(t_ref[...] * scale + shift, 0.0)
        out_ref[0] = hnew[:, :128]
        out_ref[1] = hnew[:, 128:]

    return pl.pallas_call(
        body,
        grid=(nblk,),
        in_specs=[
            pl.BlockSpec((_BLK, _H), lambda i: (i, 0)),
            pl.BlockSpec((2, _H), lambda i: (0, 0)),
            pl.BlockSpec((1, _H), lambda i: (0, 0)),
            pl.BlockSpec((1, _H), lambda i: (0, 0)),
        ],
        out_specs=pl.BlockSpec((2, _BLK, 128), lambda i: (0, i, 0)),
        out_shape=jax.ShapeDtypeStruct((2, n, 128), jnp.float32),
    )(t, stats, gamma, beta)


def _tc_bn_pool_head(t, stats, gamma, beta, batch3, l1w, l1b, l2w, l2b, n, g,
                     ncls):
    nblk = n // _BLK
    inv_n = 1.0 / n

    def body(t_ref, stats_ref, gamma_ref, beta_ref, b_ref, l1w_ref, l1b_ref,
             l2w_ref, l2b_ref, out_ref, pool_acc, cnt_acc):
        i = pl.program_id(0)

        @pl.when(i == 0)
        def _():
            pool_acc[...] = jnp.zeros_like(pool_acc)
            cnt_acc[...] = jnp.zeros_like(cnt_acc)

        mu = stats_ref[0:1, :] * inv_n
        var = stats_ref[1:2, :] * inv_n - mu * mu
        inv = lax.rsqrt(var + 1e-5)
        scale = gamma_ref[...] * inv
        shift = beta_ref[...] - mu * scale
        hnew = jnp.maximum(t_ref[...] * scale + shift, 0.0)

        bids = b_ref[0, 0, :]
        giota = lax.broadcasted_iota(jnp.int32, (g, _BLK), 0)
        onehot = (giota == bids[None, :]).astype(jnp.float32)
        pool_acc[...] += jnp.dot(onehot, hnew,
                                 preferred_element_type=jnp.float32)
        cnt_acc[...] += jnp.sum(onehot, axis=1, keepdims=True)

        @pl.when(i == nblk - 1)
        def _():
            pooled = pool_acc[...] / jnp.maximum(cnt_acc[...], 1.0)
            o = jnp.dot(pooled, l1w_ref[...],
                        preferred_element_type=jnp.float32) + l1b_ref[...]
            o = jnp.maximum(o, 0.0)
            o = jnp.dot(o, l2w_ref[...],
                        preferred_element_type=jnp.float32) + l2b_ref[...]
            m = jnp.max(o, axis=1, keepdims=True)
            ls = jnp.log(jnp.sum(jnp.exp(o - m), axis=1, keepdims=True))
            out_ref[...] = (o - m) - ls

    return pl.pallas_call(
        body,
        grid=(nblk,),
        in_specs=[
            pl.BlockSpec((_BLK, _H), lambda i: (i, 0)),
            pl.BlockSpec((2, _H), lambda i: (0, 0)),
            pl.BlockSpec((1, _H), lambda i: (0, 0)),
            pl.BlockSpec((1, _H), lambda i: (0, 0)),
            pl.BlockSpec((1, 1, _BLK), lambda i: (i, 0, 0)),
            pl.BlockSpec((_H, _H), lambda i: (0, 0)),
            pl.BlockSpec((1, _H), lambda i: (0, 0)),
            pl.BlockSpec((_H, ncls), lambda i: (0, 0)),
            pl.BlockSpec((1, ncls), lambda i: (0, 0)),
        ],
        out_specs=pl.BlockSpec((g, ncls), lambda i: (0, 0)),
        out_shape=jax.ShapeDtypeStruct((g, ncls), jnp.float32),
        scratch_shapes=[
            pltpu.VMEM((g, _H), jnp.float32),
            pltpu.VMEM((g, 1), jnp.float32),
        ],
    )(t, stats, gamma, beta, batch3, l1w, l1b, l2w, l2b)


def kernel(x, edge_index, batch, params):
    n, d = x.shape
    e = edge_index.shape[1]
    src = edge_index[0]
    dst = edge_index[1]
    gsrc_chunk = jnp.concatenate([src, src + n])
    ncls = params["lin2_W"].shape[1]
    batch3 = batch.reshape(n // _BLK, 1, _BLK)

    hs = None
    w = 128
    t = stats = None
    for li, p in enumerate(params["layers"]):
        if li == 0:
            aggs_flat = _sc_segment_add(x, src, dst, n, d, e, True)
        else:
            aggs_flat = _sc_segment_add(hs.reshape(2 * n, w), gsrc_chunk, dst,
                                        n, w, e, False)
        aggs = aggs_flat.reshape(2, n, aggs_flat.shape[1])
        b1 = p["b1"].reshape(1, _H)
        b2 = p["b2"].reshape(1, _H)
        if li == 0:
            e_row = jnp.full((1, d), 1.0, jnp.float32) + p["eps"]
            t, stats = _tc_mlp_stats_l1(x, aggs, e_row, p["W1"], b1, p["W2"],
                                        b2, n, d)
        else:
            e_row = jnp.full((1, w), 1.0, jnp.float32) + p["eps"]
            t, stats = _tc_mlp_stats(hs, aggs, e_row, p["W1"][:w],
                                     p["W1"][w:], b1, p["W2"], b2, n, w)
        if li < len(params["layers"]) - 1:
            hs = _tc_bn_relu(t, stats, p["gamma"].reshape(1, _H),
                             p["beta"].reshape(1, _H), n)

    p = params["layers"][-1]
    return _tc_bn_pool_head(t, stats, p["gamma"].reshape(1, _H),
                            p["beta"].reshape(1, _H), batch3,
                            params["lin1_W"],
                            params["lin1_b"].reshape(1, _H),
                            params["lin2_W"],
                            params["lin2_b"].reshape(1, ncls), n, _G, ncls)
